# baseline (speedup 1.0000x reference)
import functools

import jax
import jax.numpy as jnp
from jax import lax
from jax.experimental import pallas as pl
from jax.experimental.pallas import tpu as pltpu
from jax.experimental.pallas import tpu_sc as plsc

B = 16384
R_IN = 7
R_OUT = 6
D = 128
LANES = 16

_info = plsc.get_sparse_core_info()
NC, NS = _info.num_cores, _info.num_subcores
NW = NC * NS
PER_W = B // NW
NCHUNK = 64
NSTEPS = PER_W // NCHUNK


def _body(x_hbm, out_hbm, in_buf, out_buf):
    wid = lax.axis_index("s") * NC + lax.axis_index("c")
    base = wid * PER_W

    def compute_one(i, carry):
        for k in range(R_OUT):
            for v in range(D // LANES):
                sl = pl.ds(v * LANES, LANES)
                out_buf[i, k, sl] = in_buf[i, k, sl] + in_buf[i, k + 1, sl]
        return carry

    for step in range(NSTEPS):
        off = base + step * NCHUNK
        pltpu.sync_copy(x_hbm.at[pl.ds(off, NCHUNK), pl.ds(0, R_IN)], in_buf)
        lax.fori_loop(0, NCHUNK, compute_one, 0)
        pltpu.sync_copy(out_buf, out_hbm.at[pl.ds(off, NCHUNK)])


def kernel(x):
    mesh = plsc.VectorSubcoreMesh(core_axis_name="c", subcore_axis_name="s")
    run = functools.partial(
        pl.kernel,
        mesh=mesh,
        out_type=jax.ShapeDtypeStruct((B, R_OUT, D), jnp.float32),
        scratch_types=[
            pltpu.VMEM((NCHUNK, R_IN, D), jnp.float32),
            pltpu.VMEM((NCHUNK, R_OUT, D), jnp.float32),
        ],
    )(_body)
    out = run(x)
    return out.reshape(B, 3, 2, D)


# speedup vs baseline: 1.0363x; 1.0363x over previous
import functools

import jax
import jax.numpy as jnp
from jax import lax
from jax.experimental import pallas as pl
from jax.experimental.pallas import tpu as pltpu
from jax.experimental.pallas import tpu_sc as plsc

B = 16384
R_IN = 7
R_OUT = 6
D = 128
LANES = 16

_info = plsc.get_sparse_core_info()
NC, NS = _info.num_cores, _info.num_subcores
NW = NC * NS
PER_W = B // NW
NCHUNK = 32
NSTEPS = PER_W // NCHUNK


def _body(x_hbm, out_hbm, in0, in1, out0, out1, si0, si1, so0, so1):
    wid = lax.axis_index("s") * NC + lax.axis_index("c")
    base = wid * PER_W
    in_bufs = (in0, in1)
    out_bufs = (out0, out1)
    in_sems = (si0, si1)
    out_sems = (so0, so1)

    def start_in(step):
        off = base + step * NCHUNK
        return pltpu.async_copy(
            x_hbm.at[pl.ds(off, NCHUNK), pl.ds(0, R_IN)],
            in_bufs[step % 2], in_sems[step % 2])

    def start_out(step):
        off = base + step * NCHUNK
        return pltpu.async_copy(
            out_bufs[step % 2], out_hbm.at[pl.ds(off, NCHUNK)],
            out_sems[step % 2])

    def make_compute(in_buf, out_buf):
        def compute_one(i, carry):
            for k in range(R_OUT):
                for v in range(D // LANES):
                    sl = pl.ds(v * LANES, LANES)
                    out_buf[i, k // 2, k % 2, sl] = (
                        in_buf[i, k, sl] + in_buf[i, k + 1, sl])
            return carry
        return compute_one

    copies_in = [start_in(0)]
    copies_out = [None, None]
    for step in range(NSTEPS):
        cur = step % 2
        if step + 1 < NSTEPS:
            copies_in.append(start_in(step + 1))
        copies_in[step].wait()
        if copies_out[cur] is not None:
            copies_out[cur].wait()
        lax.fori_loop(0, NCHUNK, make_compute(in_bufs[cur], out_bufs[cur]), 0)
        copies_out[cur] = start_out(step)
    copies_out[(NSTEPS - 1) % 2].wait()
    copies_out[NSTEPS % 2].wait()


def kernel(x):
    mesh = plsc.VectorSubcoreMesh(core_axis_name="c", subcore_axis_name="s")
    run = functools.partial(
        pl.kernel,
        mesh=mesh,
        out_type=jax.ShapeDtypeStruct((B, 3, 2, D), jnp.float32),
        scratch_types=[
            pltpu.VMEM((NCHUNK, R_IN, D), jnp.float32),
            pltpu.VMEM((NCHUNK, R_IN, D), jnp.float32),
            pltpu.VMEM((NCHUNK, 3, 2, D), jnp.float32),
            pltpu.VMEM((NCHUNK, 3, 2, D), jnp.float32),
            pltpu.SemaphoreType.DMA,
            pltpu.SemaphoreType.DMA,
            pltpu.SemaphoreType.DMA,
            pltpu.SemaphoreType.DMA,
        ],
    )(_body)
    return run(x)


# speedup vs baseline: 1.0394x; 1.0029x over previous
import functools

import jax
import jax.numpy as jnp
from jax import lax
from jax.experimental import pallas as pl
from jax.experimental.pallas import tpu as pltpu
from jax.experimental.pallas import tpu_sc as plsc

B = 16384
R_IN = 7
R_OUT = 6
D = 128
LANES = 16

_info = plsc.get_sparse_core_info()
NC, NS = _info.num_cores, _info.num_subcores
NW = NC * NS
PER_W = B // NW
NCHUNK = 32
NSTEPS = PER_W // NCHUNK


def _body(x_hbm, out_hbm, in0, in1, out0, out1, si0, si1, so0, so1):
    wid = lax.axis_index("s") * NC + lax.axis_index("c")
    base = wid * PER_W
    in_bufs = (in0, in1)
    out_bufs = (out0, out1)
    in_sems = (si0, si1)
    out_sems = (so0, so1)

    def start_in(step):
        off = base + step * NCHUNK
        return pltpu.async_copy(
            x_hbm.at[pl.ds(off, NCHUNK), pl.ds(0, 8)],
            in_bufs[step % 2], in_sems[step % 2])

    def start_out(step):
        off = base + step * NCHUNK
        return pltpu.async_copy(
            out_bufs[step % 2], out_hbm.at[pl.ds(off, NCHUNK)],
            out_sems[step % 2])

    def make_compute(in_buf, out_buf):
        def compute_one(i, carry):
            for k in range(R_OUT):
                for v in range(D // LANES):
                    sl = pl.ds(v * LANES, LANES)
                    out_buf[i, k // 2, k % 2, sl] = (
                        in_buf[i, k, sl] + in_buf[i, k + 1, sl])
            return carry
        return compute_one

    copies_in = [start_in(0)]
    copies_out = [None, None]
    for step in range(NSTEPS):
        cur = step % 2
        if step + 1 < NSTEPS:
            copies_in.append(start_in(step + 1))
        copies_in[step].wait()
        if copies_out[cur] is not None:
            copies_out[cur].wait()
        lax.fori_loop(0, NCHUNK, make_compute(in_bufs[cur], out_bufs[cur]), 0)
        copies_out[cur] = start_out(step)
    copies_out[(NSTEPS - 1) % 2].wait()
    copies_out[NSTEPS % 2].wait()


def kernel(x):
    mesh = plsc.VectorSubcoreMesh(core_axis_name="c", subcore_axis_name="s")
    run = functools.partial(
        pl.kernel,
        mesh=mesh,
        out_type=jax.ShapeDtypeStruct((B, 3, 2, D), jnp.float32),
        compiler_params=pltpu.CompilerParams(use_tc_tiling_on_sc=True),
        scratch_types=[
            pltpu.VMEM((NCHUNK, 8, D), jnp.float32),
            pltpu.VMEM((NCHUNK, 8, D), jnp.float32),
            pltpu.VMEM((NCHUNK, 3, 2, D), jnp.float32),
            pltpu.VMEM((NCHUNK, 3, 2, D), jnp.float32),
            pltpu.SemaphoreType.DMA,
            pltpu.SemaphoreType.DMA,
            pltpu.SemaphoreType.DMA,
            pltpu.SemaphoreType.DMA,
        ],
    )(_body)
    return run(x)


# speedup vs baseline: 2.3005x; 2.2134x over previous
import functools

import jax
import jax.numpy as jnp
from jax import lax
from jax.experimental import pallas as pl
from jax.experimental.pallas import tpu as pltpu
from jax.experimental.pallas import tpu_sc as plsc

B = 16384
R_IN = 7
R_OUT = 6
D = 128
LANES = 16

_info = plsc.get_sparse_core_info()
NC, NS = _info.num_cores, _info.num_subcores
NW = NC * NS
PER_W = B // NW
NCHUNK = 32
NSTEPS = PER_W // NCHUNK


def _body(xt_hbm, out_hbm, in0, in1, out0, out1, si0, si1, so0, so1):
    wid = lax.axis_index("s") * NC + lax.axis_index("c")
    base = wid * PER_W
    in_bufs = (in0, in1)
    out_bufs = (out0, out1)
    in_sems = (si0, si1)
    out_sems = (so0, so1)

    def start_in(step):
        off = base + step * NCHUNK
        return pltpu.async_copy(
            xt_hbm.at[pl.ds(0, R_IN), pl.ds(off, NCHUNK)],
            in_bufs[step % 2], in_sems[step % 2])

    def start_out(step):
        off = base + step * NCHUNK
        return pltpu.async_copy(
            out_bufs[step % 2], out_hbm.at[pl.ds(off, NCHUNK)],
            out_sems[step % 2])

    def make_compute(in_buf, out_buf):
        def compute_one(i, carry):
            for k in range(R_OUT):
                for v in range(D // LANES):
                    sl = pl.ds(v * LANES, LANES)
                    out_buf[i, k // 2, k % 2, sl] = (
                        in_buf[k, i, sl] + in_buf[k + 1, i, sl])
            return carry
        return compute_one

    copies_in = [start_in(0)]
    copies_out = [None, None]
    for step in range(NSTEPS):
        cur = step % 2
        if step + 1 < NSTEPS:
            copies_in.append(start_in(step + 1))
        copies_in[step].wait()
        if copies_out[cur] is not None:
            copies_out[cur].wait()
        lax.fori_loop(0, NCHUNK, make_compute(in_bufs[cur], out_bufs[cur]), 0)
        copies_out[cur] = start_out(step)
    copies_out[(NSTEPS - 1) % 2].wait()
    copies_out[NSTEPS % 2].wait()


def kernel(x):
    xt = jnp.transpose(x, (1, 0, 2))
    mesh = plsc.VectorSubcoreMesh(core_axis_name="c", subcore_axis_name="s")
    run = functools.partial(
        pl.kernel,
        mesh=mesh,
        out_type=jax.ShapeDtypeStruct((B, 3, 2, D), jnp.float32),
        compiler_params=pltpu.CompilerParams(use_tc_tiling_on_sc=True),
        scratch_types=[
            pltpu.VMEM((R_IN, NCHUNK, D), jnp.float32),
            pltpu.VMEM((R_IN, NCHUNK, D), jnp.float32),
            pltpu.VMEM((NCHUNK, 3, 2, D), jnp.float32),
            pltpu.VMEM((NCHUNK, 3, 2, D), jnp.float32),
            pltpu.SemaphoreType.DMA,
            pltpu.SemaphoreType.DMA,
            pltpu.SemaphoreType.DMA,
            pltpu.SemaphoreType.DMA,
        ],
    )(_body)
    return run(xt)


# speedup vs baseline: 3.7192x; 1.6167x over previous
import functools

import jax
import jax.numpy as jnp
from jax import lax
from jax.experimental import pallas as pl
from jax.experimental.pallas import tpu as pltpu
from jax.experimental.pallas import tpu_sc as plsc

B = 16384
R_IN = 7
R_OUT = 6
D = 128
LANES = 16

_info = plsc.get_sparse_core_info()
NC, NS = _info.num_cores, _info.num_subcores
NW = NC * NS
PER_W = B // NW
NCHUNK = 32
NSTEPS = PER_W // NCHUNK


def _body(xt_hbm, out_hbm, in0, in1, out0, out1, si0, si1, so0, so1):
    wid = lax.axis_index("s") * NC + lax.axis_index("c")
    base = wid * PER_W
    in_bufs = (in0, in1)
    out_bufs = (out0, out1)
    in_sems = (si0, si1)
    out_sems = (so0, so1)

    def start_in(step):
        off = base + step * NCHUNK
        return pltpu.async_copy(
            xt_hbm.at[pl.ds(0, R_IN), pl.ds(off, NCHUNK)],
            in_bufs[step % 2], in_sems[step % 2])

    def start_out(step):
        off = base + step * NCHUNK
        return pltpu.async_copy(
            out_bufs[step % 2], out_hbm.at[pl.ds(off, NCHUNK)],
            out_sems[step % 2])

    def run_compute(in_buf, out_buf):
        @plsc.parallel_loop(0, NCHUNK, unroll=2)
        def compute_one(i):
            for k in range(R_OUT):
                for v in range(D // LANES):
                    sl = pl.ds(v * LANES, LANES)
                    out_buf[i, k // 2, k % 2, sl] = (
                        in_buf[k, i, sl] + in_buf[k + 1, i, sl])

    copies_in = [start_in(0)]
    copies_out = [None, None]
    for step in range(NSTEPS):
        cur = step % 2
        if step + 1 < NSTEPS:
            copies_in.append(start_in(step + 1))
        copies_in[step].wait()
        if copies_out[cur] is not None:
            copies_out[cur].wait()
        run_compute(in_bufs[cur], out_bufs[cur])
        copies_out[cur] = start_out(step)
    copies_out[(NSTEPS - 1) % 2].wait()
    copies_out[NSTEPS % 2].wait()


def kernel(x):
    xt = jnp.transpose(x, (1, 0, 2))
    mesh = plsc.VectorSubcoreMesh(core_axis_name="c", subcore_axis_name="s")
    run = functools.partial(
        pl.kernel,
        mesh=mesh,
        out_type=jax.ShapeDtypeStruct((B, 3, 2, D), jnp.float32),
        compiler_params=pltpu.CompilerParams(use_tc_tiling_on_sc=True),
        scratch_types=[
            pltpu.VMEM((R_IN, NCHUNK, D), jnp.float32),
            pltpu.VMEM((R_IN, NCHUNK, D), jnp.float32),
            pltpu.VMEM((NCHUNK, 3, 2, D), jnp.float32),
            pltpu.VMEM((NCHUNK, 3, 2, D), jnp.float32),
            pltpu.SemaphoreType.DMA,
            pltpu.SemaphoreType.DMA,
            pltpu.SemaphoreType.DMA,
            pltpu.SemaphoreType.DMA,
        ],
    )(_body)
    return run(xt)


# speedup vs baseline: 4.9333x; 1.3264x over previous
import functools

import jax
import jax.numpy as jnp
from jax import lax
from jax.experimental import pallas as pl
from jax.experimental.pallas import tpu as pltpu
from jax.experimental.pallas import tpu_sc as plsc

B = 16384
R_IN = 7
R_OUT = 6
D = 128
LANES = 16

_info = plsc.get_sparse_core_info()
NC, NS = _info.num_cores, _info.num_subcores
NW = NC * NS
PER_W = B // NW
NCHUNK = 32
NSTEPS = PER_W // NCHUNK


def _body(xt_hbm, out_hbm, in0, in1, out0, out1, si0, si1, so0, so1):
    wid = lax.axis_index("s") * NC + lax.axis_index("c")
    base = wid * PER_W
    in_bufs = (in0, in1)
    out_bufs = (out0, out1)
    in_sems = (si0, si1)
    out_sems = (so0, so1)

    def start_in(step, par):
        off = base + step * NCHUNK
        return pltpu.async_copy(
            xt_hbm.at[pl.ds(0, R_IN), pl.ds(off, NCHUNK)],
            in_bufs[par], in_sems[par])

    def wait_in(par):
        pltpu.make_async_copy(
            xt_hbm.at[pl.ds(0, R_IN), pl.ds(0, NCHUNK)],
            in_bufs[par], in_sems[par]).wait()

    def start_out(step, par):
        off = base + step * NCHUNK
        return pltpu.async_copy(
            out_bufs[par], out_hbm.at[pl.ds(off, NCHUNK)], out_sems[par])

    def wait_out(par):
        pltpu.make_async_copy(
            out_bufs[par], out_hbm.at[pl.ds(0, NCHUNK)], out_sems[par]).wait()

    def run_compute(par):
        in_buf, out_buf = in_bufs[par], out_bufs[par]

        @plsc.parallel_loop(0, NCHUNK, unroll=4)
        def compute_one(i):
            for k in range(R_OUT):
                for v in range(D // LANES):
                    sl = pl.ds(v * LANES, LANES)
                    out_buf[i, k // 2, k % 2, sl] = (
                        in_buf[k, i, sl] + in_buf[k + 1, i, sl])

    start_in(0, 0)

    def pair_body(g, carry):
        s0 = 2 * g
        start_in(s0 + 1, 1)
        wait_in(0)

        @pl.when(g > 0)
        def _():
            wait_out(0)
        run_compute(0)
        start_out(s0, 0)

        @pl.when(s0 + 2 < NSTEPS)
        def _():
            start_in(s0 + 2, 0)
        wait_in(1)

        @pl.when(g > 0)
        def _():
            wait_out(1)
        run_compute(1)
        start_out(s0 + 1, 1)
        return carry

    lax.fori_loop(0, NSTEPS // 2, pair_body, 0)
    wait_out(0)
    wait_out(1)


def kernel(x):
    xt = jnp.transpose(x, (1, 0, 2))
    mesh = plsc.VectorSubcoreMesh(core_axis_name="c", subcore_axis_name="s")
    run = functools.partial(
        pl.kernel,
        mesh=mesh,
        out_type=jax.ShapeDtypeStruct((B, 3, 2, D), jnp.float32),
        compiler_params=pltpu.CompilerParams(use_tc_tiling_on_sc=True),
        scratch_types=[
            pltpu.VMEM((R_IN, NCHUNK, D), jnp.float32),
            pltpu.VMEM((R_IN, NCHUNK, D), jnp.float32),
            pltpu.VMEM((NCHUNK, 3, 2, D), jnp.float32),
            pltpu.VMEM((NCHUNK, 3, 2, D), jnp.float32),
            pltpu.SemaphoreType.DMA,
            pltpu.SemaphoreType.DMA,
            pltpu.SemaphoreType.DMA,
            pltpu.SemaphoreType.DMA,
        ],
    )(_body)
    return run(xt)
